# Initial kernel scaffold; baseline (speedup 1.0000x reference)
import jax
import jax.numpy as jnp
from jax.experimental import pallas as pl


def kernel(input, Wg, bg, W1, b1, W2, b2):
    raise NotImplementedError("write your pallas kernel here")


# speedup vs baseline: 2.1757x; 2.1757x over previous
import functools

import jax
import jax.numpy as jnp
from jax.experimental import pallas as pl
import jax.experimental.pallas.tpu as pltpu

D_MODEL = 1024
D_HIDDEN = 4096
N_EXP = 8
SEQ = 2048
HB = 1024
NH = D_HIDDEN // HB


def _moe_kernel(x_ref, wg_ref, bg_ref, w1_ref, b1_ref, w2_ref, b2_ref,
                out_ref, comb_ref):
    e = pl.program_id(0)
    h = pl.program_id(1)

    @pl.when((e == 0) & (h == 0))
    def _gate():
        logits = jnp.dot(x_ref[...], wg_ref[...],
                         preferred_element_type=jnp.float32) + bg_ref[...]
        col = jax.lax.broadcasted_iota(jnp.int32, logits.shape, 1)
        v1 = jnp.max(logits, axis=1, keepdims=True)
        i1 = jnp.min(jnp.where(logits == v1, col, N_EXP), axis=1,
                     keepdims=True)
        masked = jnp.where(col == i1, -jnp.inf, logits)
        v2 = jnp.max(masked, axis=1, keepdims=True)
        i2 = jnp.min(jnp.where(masked == v2, col, N_EXP), axis=1,
                     keepdims=True)
        ev2 = jnp.exp(v2 - v1)
        g1 = 1.0 / (1.0 + ev2)
        g2 = ev2 / (1.0 + ev2)
        combine = (jnp.where(col == i1, g1, 0.0)
                   + jnp.where(col == i2, g2, 0.0))
        comb_ref[...] = combine.T
        out_ref[...] = jnp.zeros_like(out_ref)

    pre = jnp.dot(x_ref[...], w1_ref[0],
                  preferred_element_type=jnp.float32) + b1_ref[0]
    hact = jax.nn.gelu(pre, approximate=True)
    w2s = jnp.sum(w2_ref[0], axis=1)
    s = jnp.dot(hact, w2s[:, None], preferred_element_type=jnp.float32)
    b2sum = jnp.sum(b2_ref[...])
    gate = comb_ref[pl.ds(e, 1), :]
    out_ref[...] += gate * (s.reshape(1, SEQ)
                            + jnp.where(h == 0, b2sum, 0.0))

    @pl.when((e == N_EXP - 1) & (h == NH - 1))
    def _finish():
        sm = out_ref[...]
        m = jnp.max(sm)
        lse = jnp.log(jnp.sum(jnp.exp(sm - m)))
        out_ref[...] = sm - m - lse


@jax.jit
def kernel(input, Wg, bg, W1, b1, W2, b2):
    B, S, D = input.shape
    xt = input.reshape(S, D)
    out = pl.pallas_call(
        _moe_kernel,
        grid=(N_EXP, NH),
        in_specs=[
            pl.BlockSpec((SEQ, D_MODEL), lambda e, h: (0, 0)),
            pl.BlockSpec((D_MODEL, N_EXP), lambda e, h: (0, 0)),
            pl.BlockSpec((1, N_EXP), lambda e, h: (0, 0)),
            pl.BlockSpec((1, D_MODEL, HB), lambda e, h: (e, 0, h)),
            pl.BlockSpec((1, 1, HB), lambda e, h: (e, 0, h)),
            pl.BlockSpec((1, HB, D_MODEL), lambda e, h: (e, h, 0)),
            pl.BlockSpec((1, 1, D_MODEL), lambda e, h: (e, 0, 0)),
        ],
        out_specs=pl.BlockSpec((1, SEQ), lambda e, h: (0, 0)),
        out_shape=jax.ShapeDtypeStruct((1, SEQ), jnp.float32),
        scratch_shapes=[pltpu.VMEM((N_EXP, SEQ), jnp.float32)],
    )(xt, Wg, bg.reshape(1, N_EXP), W1, b1.reshape(N_EXP, 1, D_HIDDEN),
      W2, b2.reshape(N_EXP, 1, D_MODEL))
    return out.reshape(B, S)
